# Initial kernel scaffold; baseline (speedup 1.0000x reference)
import jax
import jax.numpy as jnp
from jax.experimental import pallas as pl


def kernel(X, emb_tables, dense_weight):
    raise NotImplementedError("write your pallas kernel here")


# speedup vs baseline: 1.4144x; 1.4144x over previous
import jax
import jax.numpy as jnp
from jax import lax
from jax.experimental import pallas as pl
from jax.experimental.pallas import tpu as pltpu
from jax.experimental.pallas import tpu_sc as plsc

_B = 16384
_NS = 26
_ND = 13
_NF = _NS + _ND
_V = 100000
_L = 16
_NC = 2
_NSUB = 16
_NW = _NC * _NSUB
_RPT = _B // _NW
_NG = _RPT // _L


def _body(xt_hbm, t_hbm, w_hbm, out_hbm, xv, idxv, gv, accv, wv, sem):
    wid = lax.axis_index("s") * _NC + lax.axis_index("c")
    base = wid * _RPT
    pltpu.sync_copy(xt_hbm.at[:, pl.ds(base, _RPT)], xv)
    pltpu.sync_copy(w_hbm, wv)

    def build(i, carry):
        off = i * _L
        for f in range(_NS):
            v = xv[f, pl.ds(off, _L)]
            idxv[pl.ds(f * _RPT + off, _L)] = v.astype(jnp.int32) + f * _V
        return carry

    lax.fori_loop(0, _NG, build, 0)

    pltpu.async_copy(t_hbm.at[idxv], gv, sem).wait()

    def reduce(i, carry):
        off = i * _L
        acc = jnp.zeros((_L,), jnp.float32)
        for f in range(_NS):
            acc = acc + gv[pl.ds(f * _RPT + off, _L)]
        for d in range(_ND):
            acc = acc + xv[_NS + d, pl.ds(off, _L)] * wv[d]
        accv[pl.ds(off, _L)] = acc
        return carry

    lax.fori_loop(0, _NG, reduce, 0)

    pltpu.sync_copy(accv, out_hbm.at[pl.ds(base, _RPT)])


def kernel(X, emb_tables, dense_weight):
    xt = X.T
    t_flat = emb_tables.reshape(_NS * _V)
    w16 = jnp.broadcast_to(dense_weight.reshape(_ND, 1), (_ND, _L))
    mesh = plsc.VectorSubcoreMesh(core_axis_name="c", subcore_axis_name="s")
    out = pl.kernel(
        _body,
        out_type=jax.ShapeDtypeStruct((_B,), jnp.float32),
        mesh=mesh,
        scratch_types=[
            pltpu.VMEM((_NF, _RPT), jnp.float32),
            pltpu.VMEM((_NS * _RPT,), jnp.int32),
            pltpu.VMEM((_NS * _RPT,), jnp.float32),
            pltpu.VMEM((_RPT,), jnp.float32),
            pltpu.VMEM((_ND, _L), jnp.float32),
            pltpu.SemaphoreType.DMA,
        ],
    )(xt, t_flat, w16)
    return out.reshape(_B, 1)
